# Initial kernel scaffold; baseline (speedup 1.0000x reference)
import jax
import jax.numpy as jnp
from jax.experimental import pallas as pl


def kernel(query, values, prev_att, params):
    raise NotImplementedError("write your pallas kernel here")


# speedup vs baseline: 7.6477x; 7.6477x over previous
import jax
import jax.numpy as jnp
from jax import lax
from jax.experimental import pallas as pl
from jax.experimental.pallas import tpu as pltpu


def _bf16_rne(x):
    u = lax.bitcast_convert_type(x, jnp.int32)
    r = (u + jnp.int32(0x7FFF) + ((u >> 16) & 1)) & jnp.int32(-65536)
    return lax.bitcast_convert_type(r, jnp.float32)


def _sortable(x):
    xi = lax.bitcast_convert_type(x, jnp.int32)
    return xi ^ ((xi >> 31) & jnp.int32(0x7FFFFFFF))


def _topk_mask(v, s, k):
    tau = _kth_largest(v, k)
    gt = v > tau
    eq = v == tau
    need = k - jnp.sum(gt.astype(jnp.int32), axis=-1, keepdims=True)
    N = v.shape[-1]
    iota = lax.broadcasted_iota(jnp.int32, v.shape, v.ndim - 1)
    eqi = eq.astype(jnp.int32)
    c = jnp.zeros(v.shape[:-1] + (1,), jnp.int32)
    b = N // 2
    while b >= 1:
        cnt_lt = jnp.sum(jnp.where(iota < (c + b), eqi, 0),
                         axis=-1, keepdims=True)
        c = jnp.where(cnt_lt < need, c + b, c)
        b //= 2
    keep = gt | (eq & (iota <= c))
    return jnp.where(keep, s, 0.0)


def _kth_largest(v, k):
    p0 = jnp.full(v.shape[:-1] + (1,), jnp.int32(-(2**31)))

    def body(i, p):
        c = p + (jnp.int32(1) << (jnp.int32(31) - i))
        cnt = jnp.sum((v >= c).astype(jnp.int32), axis=-1, keepdims=True)
        return jnp.where(cnt >= k, c, p)

    return lax.fori_loop(0, 32, body, p0)


def _scores_body(values_ref, cw3_ref, prev_ref, query_ref, w1t_ref, w2t_ref,
                 qb_ref, locp_ref, vw_ref, vb_ref, out_ref):
    B, Tb, H = values_ref.shape
    prev = prev_ref[...]
    q = query_ref[...]
    Rb = min(16, B)
    for h in range(3):
        cw = cw3_ref[h]
        convo = lax.dot_general(prev, cw, (((1,), (1,)), ((), ())),
                                preferred_element_type=jnp.float32)
        qt = jnp.dot(q, w2t_ref[h],
                     preferred_element_type=jnp.float32) + qb_ref[h][None, :]
        vwb = jnp.broadcast_to(vw_ref[h][:, None], (H, 128))
        for rb in range(0, B, Rb):
            v = values_ref[rb:rb + Rb].reshape(Rb * Tb, H)
            p1 = jnp.dot(v, w1t_ref[h],
                         preferred_element_type=jnp.float32).reshape(Rb, Tb, H)
            s1 = (p1 + qt[rb:rb + Rb, None, :]
                  + convo[rb:rb + Rb, :, None] * locp_ref[h][None, None, :])
            z = jnp.tanh(s1).reshape(Rb * Tb, H)
            sc = jnp.dot(z, vwb,
                         preferred_element_type=jnp.float32)
            sc = sc.reshape(Rb, Tb, 128)[:, :, 0] + vb_ref[h]
            out_ref[h, rb:rb + Rb] = sc


def _scores_call(values, cw3, prev2, query, w1t, w2t, qb, locp, vw, vb):
    B, T, H = values.shape
    Tb = 128 if T % 128 == 0 else T
    return pl.pallas_call(
        _scores_body,
        grid=(T // Tb,),
        in_specs=[
            pl.BlockSpec((B, Tb, H), lambda i: (0, i, 0)),
            pl.BlockSpec((3, Tb, T), lambda i: (0, i, 0)),
            pl.BlockSpec((B, T), lambda i: (0, 0)),
            pl.BlockSpec((B, H), lambda i: (0, 0)),
            pl.BlockSpec((3, H, H), lambda i: (0, 0, 0)),
            pl.BlockSpec((3, H, H), lambda i: (0, 0, 0)),
            pl.BlockSpec((3, H), lambda i: (0, 0)),
            pl.BlockSpec((3, H), lambda i: (0, 0)),
            pl.BlockSpec((3, H), lambda i: (0, 0)),
            pl.BlockSpec(memory_space=pltpu.SMEM),
        ],
        out_specs=pl.BlockSpec((3, B, Tb), lambda i: (0, 0, i)),
        out_shape=jax.ShapeDtypeStruct((3, B, T), jnp.float32),
    )(values, cw3, prev2, query, w1t, w2t, qb, locp, vw, vb)


def _mask_body(s3_ref, wmix_ref, smask_ref, sig_ref, *, k):
    s3 = s3_ref[...]
    v3 = _sortable(s3)
    m3 = _topk_mask(v3, s3, k)
    m3b = _bf16_rne(m3)
    w0 = _bf16_rne(wmix_ref[0])
    w1 = _bf16_rne(wmix_ref[1])
    w2 = _bf16_rne(wmix_ref[2])
    cmb = m3b[0] * w0 + m3b[1] * w1 + m3b[2] * w2 + wmix_ref[3]
    vc = _sortable(cmb)
    sm = _topk_mask(vc, cmb, k)
    smask_ref[...] = sm
    sig_ref[...] = 1.0 / (1.0 + jnp.exp(-sm))


def _mask_call(scores3, wmix4, k):
    import functools
    _, B, T = scores3.shape
    return pl.pallas_call(
        functools.partial(_mask_body, k=k),
        in_specs=[
            pl.BlockSpec((3, B, T), lambda: (0, 0, 0)),
            pl.BlockSpec(memory_space=pltpu.SMEM),
        ],
        out_specs=[
            pl.BlockSpec((B, T), lambda: (0, 0)),
            pl.BlockSpec((B, T), lambda: (0, 0)),
        ],
        out_shape=[
            jax.ShapeDtypeStruct((B, T), jnp.float32),
            jax.ShapeDtypeStruct((B, T), jnp.float32),
        ],
    )(scores3, wmix4)


def _finalize_body(values_ref, sig_ref, ctx_ref, att_ref):
    sg = sig_ref[...]
    colsum = jnp.sum(sg, axis=0, keepdims=True)
    att = sg / colsum
    att_ref[...] = att
    v = values_ref[...]
    partial = jnp.sum(att[:, :, None] * v, axis=1)

    @pl.when(pl.program_id(0) == 0)
    def _():
        ctx_ref[...] = jnp.zeros_like(ctx_ref)

    ctx_ref[...] += partial


def _finalize_call(values, sig):
    B, T, H = values.shape
    Tb = 128 if T % 128 == 0 else T
    return pl.pallas_call(
        _finalize_body,
        grid=(T // Tb,),
        in_specs=[
            pl.BlockSpec((B, Tb, H), lambda i: (0, i, 0)),
            pl.BlockSpec((B, Tb), lambda i: (0, i)),
        ],
        out_specs=[
            pl.BlockSpec((B, H), lambda i: (0, 0)),
            pl.BlockSpec((B, Tb), lambda i: (0, i)),
        ],
        out_shape=[
            jax.ShapeDtypeStruct((B, H), jnp.float32),
            jax.ShapeDtypeStruct((B, T), jnp.float32),
        ],
    )(values, sig)


def kernel(query, values, prev_att, params):
    B, T, H = values.shape
    heads = params['heads']
    mid = heads[0]['conv_w'].shape[-1] // 2
    k = T * 2 // 3

    prev2 = prev_att[..., 0]
    cw3 = jnp.stack([hp['conv_w'][:, :, mid] for hp in heads])
    w1t = jnp.stack([hp['W1_w'].T for hp in heads])
    w2t = jnp.stack([hp['W2_w'].T for hp in heads])
    qb = jnp.stack([hp['W1_b'] + hp['W2_b'] for hp in heads])
    locp = jnp.stack([hp['loc_proj_w'][:, 0] for hp in heads])
    vw = jnp.stack([hp['V_w'][0] for hp in heads])
    vb = jnp.stack([hp['V_b'][0] for hp in heads])
    wmix4 = jnp.concatenate([params['W_w'][0], params['W_b']])

    scores3 = _scores_call(values, cw3, prev2, query, w1t, w2t, qb, locp,
                           vw, vb)
    smask, sig = _mask_call(scores3, wmix4, k)
    ctx, att = _finalize_call(values, sig)
    return ctx, att[..., None], smask[..., None]
